# Initial kernel scaffold; baseline (speedup 1.0000x reference)
import functools

import jax
import jax.numpy as jnp
from jax import lax
from jax.experimental import pallas as pl
from jax.experimental.pallas import tpu as pltpu
from jax.experimental.pallas import tpu_sc as plsc

_N = 10000
_E = 320000
_CHUNK = 128
_NCHUNK = _E // _CHUNK
_NS = 16
_CPT = -(-_NCHUNK // _NS)
_RPT = _N // _NS
_ZROWS = 125
_BM = 400


def _leaky(t):
    return jnp.where(t >= 0, t, 0.01 * t)


def _deg_body(dst_hbm, out_hbm, didx, ones, zbuf, acc):
    c = lax.axis_index("c")
    s = lax.axis_index("s")

    def fillz(i, carry):
        zbuf[pl.ds(i * 16, 16)] = jnp.zeros((16,), jnp.float32)
        return carry

    lax.fori_loop(0, 1000 // 16, fillz, 0)

    def fill1(i, carry):
        ones[pl.ds(i * 16, 16)] = jnp.ones((16,), jnp.float32)
        return carry

    lax.fori_loop(0, _CHUNK // 16, fill1, 0)

    @pl.when(s < 10)
    def _():
        pltpu.sync_copy(zbuf, acc.at[pl.ds(s * 1000, 1000)])

    plsc.subcore_barrier()

    def chunk(j, carry):
        ch = s + _NS * j

        @pl.when(ch < _NCHUNK)
        def _():
            pltpu.sync_copy(dst_hbm.at[pl.ds(ch * _CHUNK, _CHUNK)], didx)
            pltpu.sync_copy(ones, acc.at[didx], add=True)

        return carry

    lax.fori_loop(0, _CPT, chunk, 0)
    plsc.subcore_barrier()

    @pl.when(jnp.logical_and(s == 0, c == 0))
    def _():
        pltpu.sync_copy(acc, out_hbm)


_deg = functools.partial(
    pl.kernel,
    out_type=jax.ShapeDtypeStruct((_N,), jnp.float32),
    mesh=plsc.VectorSubcoreMesh(core_axis_name="c", subcore_axis_name="s"),
    scratch_types=[
        pltpu.VMEM((_CHUNK,), jnp.int32),
        pltpu.VMEM((_CHUNK,), jnp.float32),
        pltpu.VMEM((1000,), jnp.float32),
        pltpu.VMEM_SHARED((_N,), jnp.float32),
    ],
)(_deg_body)


def _make_scatter(hw):
    def body(y_hbm, src_hbm, dst_hbm, out0, out1,
             sidx, gidx, didx, rowbuf, zbuf, acc, sem):
        c = lax.axis_index("c")
        s = lax.axis_index("s")

        def fillz(i, carry):
            for l in range(hw // 16):
                zbuf[i, pl.ds(l * 16, 16)] = jnp.zeros((16,), jnp.float32)
            return carry

        lax.fori_loop(0, _ZROWS, fillz, 0)

        for k in range(_RPT // _ZROWS):
            pltpu.sync_copy(zbuf, acc.at[pl.ds(s * _RPT + k * _ZROWS, _ZROWS)])

        plsc.subcore_barrier()

        def chunk(j, carry):
            ch = s + _NS * j

            @pl.when(ch < _NCHUNK)
            def _():
                pltpu.sync_copy(src_hbm.at[pl.ds(ch * _CHUNK, _CHUNK)], sidx)
                pltpu.sync_copy(dst_hbm.at[pl.ds(ch * _CHUNK, _CHUNK)], didx)
                for t in range(_CHUNK // 16):
                    sv = sidx[pl.ds(t * 16, 16)]
                    gidx[pl.ds(t * 16, 16)] = sv * 2 + c
                pltpu.async_copy(y_hbm.at[gidx], rowbuf, sem).wait()
                pltpu.sync_copy(rowbuf, acc.at[didx], add=True)

            return carry

        lax.fori_loop(0, _CPT, chunk, 0)
        plsc.subcore_barrier()

        @pl.when(c == 0)
        def _():
            pltpu.sync_copy(acc.at[pl.ds(s * _RPT, _RPT)],
                            out0.at[pl.ds(s * _RPT, _RPT)])

        @pl.when(c == 1)
        def _():
            pltpu.sync_copy(acc.at[pl.ds(s * _RPT, _RPT)],
                            out1.at[pl.ds(s * _RPT, _RPT)])

    return functools.partial(
        pl.kernel,
        out_type=(jax.ShapeDtypeStruct((_N, hw), jnp.float32),
                  jax.ShapeDtypeStruct((_N, hw), jnp.float32)),
        mesh=plsc.VectorSubcoreMesh(core_axis_name="c", subcore_axis_name="s"),
        scratch_types=[
            pltpu.VMEM((_CHUNK,), jnp.int32),
            pltpu.VMEM((_CHUNK,), jnp.int32),
            pltpu.VMEM((_CHUNK,), jnp.int32),
            pltpu.VMEM((_CHUNK, hw), jnp.float32),
            pltpu.VMEM((_ZROWS, hw), jnp.float32),
            pltpu.VMEM_SHARED((_N, hw), jnp.float32),
            pltpu.SemaphoreType.DMA,
        ],
    )(body)


_scatter128 = _make_scatter(128)
_scatter64 = _make_scatter(64)


def _dense_body(x_ref, w1_ref, b1_ref, w2_ref, b2_ref, wc1_ref, deg_ref, y_ref):
    h = _leaky(jnp.dot(x_ref[...], w1_ref[...],
                       preferred_element_type=jnp.float32) + b1_ref[...])
    h = _leaky(jnp.dot(h, w2_ref[...],
                       preferred_element_type=jnp.float32) + b2_ref[...])
    xw = jnp.dot(h, wc1_ref[...], preferred_element_type=jnp.float32)
    dis = lax.rsqrt(deg_ref[...] + 1.0)
    y_ref[...] = xw * dis


def _mid_body(agg0_ref, agg1_ref, y_ref, deg_ref, bc1_ref, wc2_ref,
              e1_ref, y2_ref):
    dis = lax.rsqrt(deg_ref[...] + 1.0)
    agg = jnp.concatenate([agg0_ref[...], agg1_ref[...]], axis=1)
    e1 = _leaky(dis * (agg + y_ref[...]) + bc1_ref[...])
    e1_ref[...] = e1
    y2_ref[...] = jnp.dot(e1, wc2_ref[...],
                          preferred_element_type=jnp.float32) * dis


def _final_body(agg0_ref, agg1_ref, y2_ref, deg_ref, bc2_ref, e2_ref):
    dis = lax.rsqrt(deg_ref[...] + 1.0)
    agg = jnp.concatenate([agg0_ref[...], agg1_ref[...]], axis=1)
    e2_ref[...] = _leaky(dis * (agg + y2_ref[...]) + bc2_ref[...])


def _row_spec(w):
    return pl.BlockSpec((_BM, w), lambda i: (i, 0))


def _full_spec(h, w):
    return pl.BlockSpec((h, w), lambda i: (0, 0))


_GRID = _N // _BM

_dense = pl.pallas_call(
    _dense_body,
    grid=(_GRID,),
    in_specs=[_row_spec(128), _full_spec(128, 256), _full_spec(1, 256),
              _full_spec(256, 256), _full_spec(1, 256), _full_spec(256, 256),
              _row_spec(1)],
    out_specs=_row_spec(256),
    out_shape=jax.ShapeDtypeStruct((_N, 256), jnp.float32),
)

_mid = pl.pallas_call(
    _mid_body,
    grid=(_GRID,),
    in_specs=[_row_spec(128), _row_spec(128), _row_spec(256), _row_spec(1),
              _full_spec(1, 256), _full_spec(256, 128)],
    out_specs=(_row_spec(256), _row_spec(128)),
    out_shape=(jax.ShapeDtypeStruct((_N, 256), jnp.float32),
               jax.ShapeDtypeStruct((_N, 128), jnp.float32)),
)

_final = pl.pallas_call(
    _final_body,
    grid=(_GRID,),
    in_specs=[_row_spec(64), _row_spec(64), _row_spec(128), _row_spec(1),
              _full_spec(1, 128)],
    out_specs=_row_spec(128),
    out_shape=jax.ShapeDtypeStruct((_N, 128), jnp.float32),
)


def kernel(x, edge_index, W1, b1, W2, b2, Wc1, bc1, Wc2, bc2, prev1, prev2):
    src = edge_index[0]
    dst = edge_index[1]
    deg = _deg(dst)
    deg_col = deg.reshape(_N, 1)
    y1 = _dense(x, W1, b1.reshape(1, -1), W2, b2.reshape(1, -1), Wc1, deg_col)
    a0, a1 = _scatter128(y1.reshape(2 * _N, 128), src, dst)
    e1, y2 = _mid(a0, a1, y1, deg_col, bc1.reshape(1, -1), Wc2)
    c0, c1 = _scatter64(y2.reshape(2 * _N, 64), src, dst)
    e2 = _final(c0, c1, y2, deg_col, bc2.reshape(1, -1))
    return (e1, e2)


# speedup vs baseline: 11.7841x; 11.7841x over previous
import functools

import jax
import jax.numpy as jnp
from jax import lax
from jax.experimental import pallas as pl
from jax.experimental.pallas import tpu as pltpu
from jax.experimental.pallas import tpu_sc as plsc

_N = 10000
_NPAD = 10240
_E = 320000
_CHUNK = 128
_NCHUNK = _E // _CHUNK
_NS = 16
_CPT = -(-_NCHUNK // _NS)
_RPT = _NPAD // _NS
_ZROWS = 128
_BM = 400


def _leaky(t):
    return jnp.where(t >= 0, t, 0.01 * t)


def _deg_body(dst_hbm, out_hbm, didx, ones, zbuf, acc):
    c = lax.axis_index("c")
    s = lax.axis_index("s")

    def fillz(i, carry):
        zbuf[pl.ds(i * 16, 16)] = jnp.zeros((16,), jnp.float32)
        return carry

    lax.fori_loop(0, _RPT // 16, fillz, 0)

    def fill1(i, carry):
        ones[pl.ds(i * 16, 16)] = jnp.ones((16,), jnp.float32)
        return carry

    lax.fori_loop(0, _CHUNK // 16, fill1, 0)

    pltpu.sync_copy(zbuf, acc.at[pl.ds(s * _RPT, _RPT)])

    plsc.subcore_barrier()

    def chunk(j, carry):
        ch = s + _NS * j

        @pl.when(ch < _NCHUNK)
        def _():
            pltpu.sync_copy(dst_hbm.at[pl.ds(ch * _CHUNK, _CHUNK)], didx)
            pltpu.sync_copy(ones, acc.at[didx], add=True)

        return carry

    lax.fori_loop(0, _CPT, chunk, 0)
    plsc.subcore_barrier()

    @pl.when(jnp.logical_and(s == 0, c == 0))
    def _():
        pltpu.sync_copy(acc, out_hbm)


_deg = functools.partial(
    pl.kernel,
    out_type=jax.ShapeDtypeStruct((_NPAD,), jnp.float32),
    mesh=plsc.VectorSubcoreMesh(core_axis_name="c", subcore_axis_name="s"),
    scratch_types=[
        pltpu.VMEM((_CHUNK,), jnp.int32),
        pltpu.VMEM((_CHUNK,), jnp.float32),
        pltpu.VMEM((_RPT,), jnp.float32),
        pltpu.VMEM_SHARED((_NPAD,), jnp.float32),
    ],
)(_deg_body)


def _make_scatter(hw):
    def body(y_hbm, src_hbm, dst_hbm, out0, out1,
             sidx, gidx, didx, rowbuf, zbuf, acc, sem):
        c = lax.axis_index("c")
        s = lax.axis_index("s")

        def fillz(i, carry):
            for l in range(hw // 16):
                zbuf[i, pl.ds(l * 16, 16)] = jnp.zeros((16,), jnp.float32)
            return carry

        lax.fori_loop(0, _ZROWS, fillz, 0)

        for k in range(_RPT // _ZROWS):
            pltpu.sync_copy(zbuf, acc.at[pl.ds(s * _RPT + k * _ZROWS, _ZROWS)])

        plsc.subcore_barrier()

        def chunk(j, carry):
            ch = s + _NS * j

            @pl.when(ch < _NCHUNK)
            def _():
                pltpu.sync_copy(src_hbm.at[pl.ds(ch * _CHUNK, _CHUNK)], sidx)
                pltpu.sync_copy(dst_hbm.at[pl.ds(ch * _CHUNK, _CHUNK)], didx)
                for t in range(_CHUNK // 16):
                    sv = sidx[pl.ds(t * 16, 16)]
                    gidx[pl.ds(t * 16, 16)] = sv * 2 + c
                pltpu.async_copy(y_hbm.at[gidx], rowbuf, sem).wait()
                pltpu.sync_copy(rowbuf, acc.at[didx], add=True)

            return carry

        lax.fori_loop(0, _CPT, chunk, 0)
        plsc.subcore_barrier()

        @pl.when(c == 0)
        def _():
            pltpu.sync_copy(acc.at[pl.ds(s * _RPT, _RPT)],
                            out0.at[pl.ds(s * _RPT, _RPT)])

        @pl.when(c == 1)
        def _():
            pltpu.sync_copy(acc.at[pl.ds(s * _RPT, _RPT)],
                            out1.at[pl.ds(s * _RPT, _RPT)])

    return functools.partial(
        pl.kernel,
        out_type=(jax.ShapeDtypeStruct((_NPAD, hw), jnp.float32),
                  jax.ShapeDtypeStruct((_NPAD, hw), jnp.float32)),
        mesh=plsc.VectorSubcoreMesh(core_axis_name="c", subcore_axis_name="s"),
        scratch_types=[
            pltpu.VMEM((_CHUNK,), jnp.int32),
            pltpu.VMEM((_CHUNK,), jnp.int32),
            pltpu.VMEM((_CHUNK,), jnp.int32),
            pltpu.VMEM((_CHUNK, hw), jnp.float32),
            pltpu.VMEM((_ZROWS, hw), jnp.float32),
            pltpu.VMEM_SHARED((_NPAD, hw), jnp.float32),
            pltpu.SemaphoreType.DMA,
        ],
    )(body)


_scatter128 = _make_scatter(128)


def _scatter_edges_body(y_hbm, src_hbm, dst_hbm, out0, out1,
                        sidx, didx, rowbuf, zbuf, acc, sem):
    c = lax.axis_index("c")
    s = lax.axis_index("s")

    def fillz(i, carry):
        for l in range(128 // 16):
            zbuf[i, pl.ds(l * 16, 16)] = jnp.zeros((16,), jnp.float32)
        return carry

    lax.fori_loop(0, _ZROWS, fillz, 0)

    for k in range(_RPT // _ZROWS):
        pltpu.sync_copy(zbuf, acc.at[pl.ds(s * _RPT + k * _ZROWS, _ZROWS)])

    plsc.subcore_barrier()

    def chunk(j, carry):
        half = s + _NS * j

        @pl.when(half < _NCHUNK // 2)
        def _():
            ch = 2 * half + c
            pltpu.sync_copy(src_hbm.at[pl.ds(ch * _CHUNK, _CHUNK)], sidx)
            pltpu.sync_copy(dst_hbm.at[pl.ds(ch * _CHUNK, _CHUNK)], didx)
            pltpu.async_copy(y_hbm.at[sidx], rowbuf, sem).wait()
            pltpu.sync_copy(rowbuf, acc.at[didx], add=True)

        return carry

    lax.fori_loop(0, -(-(_NCHUNK // 2) // _NS), chunk, 0)
    plsc.subcore_barrier()

    @pl.when(c == 0)
    def _():
        pltpu.sync_copy(acc.at[pl.ds(s * _RPT, _RPT)],
                        out0.at[pl.ds(s * _RPT, _RPT)])

    @pl.when(c == 1)
    def _():
        pltpu.sync_copy(acc.at[pl.ds(s * _RPT, _RPT)],
                        out1.at[pl.ds(s * _RPT, _RPT)])


_scatter_edges = functools.partial(
    pl.kernel,
    out_type=(jax.ShapeDtypeStruct((_NPAD, 128), jnp.float32),
              jax.ShapeDtypeStruct((_NPAD, 128), jnp.float32)),
    mesh=plsc.VectorSubcoreMesh(core_axis_name="c", subcore_axis_name="s"),
    scratch_types=[
        pltpu.VMEM((_CHUNK,), jnp.int32),
        pltpu.VMEM((_CHUNK,), jnp.int32),
        pltpu.VMEM((_CHUNK, 128), jnp.float32),
        pltpu.VMEM((_ZROWS, 128), jnp.float32),
        pltpu.VMEM_SHARED((_NPAD, 128), jnp.float32),
        pltpu.SemaphoreType.DMA,
    ],
)(_scatter_edges_body)


def _dense_body(x_ref, w1_ref, b1_ref, w2_ref, b2_ref, wc1_ref, deg_ref, y_ref):
    h = _leaky(jnp.dot(x_ref[...], w1_ref[...],
                       preferred_element_type=jnp.float32) + b1_ref[...])
    h = _leaky(jnp.dot(h, w2_ref[...],
                       preferred_element_type=jnp.float32) + b2_ref[...])
    xw = jnp.dot(h, wc1_ref[...], preferred_element_type=jnp.float32)
    dis = lax.rsqrt(deg_ref[...] + 1.0)
    y_ref[...] = xw * dis


def _mid_body(agg0_ref, agg1_ref, y_ref, deg_ref, bc1_ref, wc2_ref,
              e1_ref, y2_ref):
    dis = lax.rsqrt(deg_ref[...] + 1.0)
    agg = jnp.concatenate([agg0_ref[...], agg1_ref[...]], axis=1)
    e1 = _leaky(dis * (agg + y_ref[...]) + bc1_ref[...])
    e1_ref[...] = e1
    y2_ref[...] = jnp.dot(e1, wc2_ref[...],
                          preferred_element_type=jnp.float32) * dis


def _final_body(agg0_ref, agg1_ref, y2_ref, deg_ref, bc2_ref, e2_ref):
    dis = lax.rsqrt(deg_ref[...] + 1.0)
    agg = agg0_ref[...] + agg1_ref[...]
    e2_ref[...] = _leaky(dis * (agg + y2_ref[...]) + bc2_ref[...])


def _row_spec(w):
    return pl.BlockSpec((_BM, w), lambda i: (i, 0))


def _full_spec(h, w):
    return pl.BlockSpec((h, w), lambda i: (0, 0))


_GRID = _N // _BM

_dense = pl.pallas_call(
    _dense_body,
    grid=(_GRID,),
    in_specs=[_row_spec(128), _full_spec(128, 256), _full_spec(1, 256),
              _full_spec(256, 256), _full_spec(1, 256), _full_spec(256, 256),
              _row_spec(1)],
    out_specs=_row_spec(256),
    out_shape=jax.ShapeDtypeStruct((_N, 256), jnp.float32),
)

_mid = pl.pallas_call(
    _mid_body,
    grid=(_GRID,),
    in_specs=[_row_spec(128), _row_spec(128), _row_spec(256), _row_spec(1),
              _full_spec(1, 256), _full_spec(256, 128)],
    out_specs=(_row_spec(256), _row_spec(128)),
    out_shape=(jax.ShapeDtypeStruct((_N, 256), jnp.float32),
               jax.ShapeDtypeStruct((_N, 128), jnp.float32)),
)

_final = pl.pallas_call(
    _final_body,
    grid=(_GRID,),
    in_specs=[_row_spec(128), _row_spec(128), _row_spec(128), _row_spec(1),
              _full_spec(1, 128)],
    out_specs=_row_spec(128),
    out_shape=jax.ShapeDtypeStruct((_N, 128), jnp.float32),
)


def kernel(x, edge_index, W1, b1, W2, b2, Wc1, bc1, Wc2, bc2, prev1, prev2):
    src = edge_index[0]
    dst = edge_index[1]
    deg = _deg(dst)[:_N]
    deg_col = deg.reshape(_N, 1)
    y1 = _dense(x, W1, b1.reshape(1, -1), W2, b2.reshape(1, -1), Wc1, deg_col)
    a0, a1 = _scatter128(y1.reshape(2 * _N, 128), src, dst)
    e1, y2 = _mid(a0[:_N], a1[:_N], y1, deg_col, bc1.reshape(1, -1), Wc2)
    c0, c1 = _scatter_edges(y2, src, dst)
    e2 = _final(c0[:_N], c1[:_N], y2, deg_col, bc2.reshape(1, -1))
    return (e1, e2)
